# baseline (speedup 1.0000x reference)
import jax
import jax.numpy as jnp
from jax.experimental import pallas as pl
from jax.experimental.pallas import tpu as pltpu

NSUB = 2


def _chain(x, W1, b1, W2, b2, E, Et, Wd1, bd1, Wd2, bd2):
    h = jnp.maximum(
        jnp.dot(x, W1, preferred_element_type=jnp.float32) + b1, 0.0)
    z = jnp.maximum(
        jnp.dot(h, W2, preferred_element_type=jnp.float32) + b2, 0.0)
    sim = jnp.dot(z, E, preferred_element_type=jnp.float32)
    z_sq = jnp.sum(z * z, axis=1, keepdims=True)
    e_sq = jnp.sum(E * E, axis=0, keepdims=True)
    dist = z_sq + e_sq - 2.0 * sim
    idx = jnp.argmin(dist, axis=1)
    k_iota = jax.lax.broadcasted_iota(jnp.int32, dist.shape, 1)
    onehot = (k_iota == idx[:, None]).astype(jnp.float32)
    quant = jnp.dot(onehot, Et, preferred_element_type=jnp.float32)
    q = z + (quant - z)
    hd = jnp.maximum(
        jnp.dot(q, Wd1, preferred_element_type=jnp.float32) + bd1, 0.0)
    return jnp.dot(hd, Wd2, preferred_element_type=jnp.float32) + bd2


def _body(x_ref, W1_ref, b1_ref, W2_ref, b2_ref, E_ref, Et_ref,
          Wd1_ref, bd1_ref, Wd2_ref, bd2_ref, out_ref):
    TB = x_ref.shape[0]
    S = TB // NSUB
    E = E_ref[...]
    Et = Et_ref[...]
    W1 = W1_ref[...]
    W2 = W2_ref[...]
    Wd1 = Wd1_ref[...]
    Wd2 = Wd2_ref[...]
    b1 = b1_ref[...]
    b2 = b2_ref[...]
    bd1 = bd1_ref[...]
    bd2 = bd2_ref[...]
    for s in range(NSUB):
        rows = pl.ds(s * S, S)
        out_ref[rows, :] = _chain(
            x_ref[rows, :], W1, b1, W2, b2, E, Et, Wd1, bd1, Wd2, bd2)


@jax.jit
def kernel(x, W1, b1, W2, b2, E, Wd1, bd1, Wd2, bd2):
    B, D = x.shape
    L, K = E.shape
    Dh = W1.shape[1]
    TB = min(2048, B)
    grid = (B // TB,)

    def batch_map(i):
        return (i, 0)

    def const_map(i):
        return (0, 0)

    full = lambda shape: pl.BlockSpec(shape, const_map)
    out = pl.pallas_call(
        _body,
        grid=grid,
        in_specs=[
            pl.BlockSpec((TB, D), batch_map),
            full((D, Dh)),
            full((1, Dh)),
            full((Dh, L)),
            full((1, L)),
            full((L, K)),
            full((K, L)),
            full((L, Dh)),
            full((1, Dh)),
            full((Dh, D)),
            full((1, D)),
        ],
        out_specs=pl.BlockSpec((TB, D), batch_map),
        out_shape=jax.ShapeDtypeStruct((B, D), jnp.float32),
        compiler_params=pltpu.CompilerParams(
            dimension_semantics=("arbitrary",),
        ),
    )(x, W1, b1.reshape(1, -1), W2, b2.reshape(1, -1), E,
      E.T,
      Wd1, bd1.reshape(1, -1), Wd2, bd2.reshape(1, -1))
    return out


# speedup vs baseline: 1.0499x; 1.0499x over previous
import jax
import jax.numpy as jnp
from jax.experimental import pallas as pl
from jax.experimental.pallas import tpu as pltpu

NSUB = 1


def _chain(x, W1, b1, W2, b2, E, Wd1, bd1, Wd2, bd2):
    h = jnp.maximum(
        jnp.dot(x, W1, preferred_element_type=jnp.float32) + b1, 0.0)
    z = jnp.maximum(
        jnp.dot(h, W2, preferred_element_type=jnp.float32) + b2, 0.0)
    sim = jnp.dot(z, E, preferred_element_type=jnp.float32)
    z_sq = jnp.sum(z * z, axis=1, keepdims=True)
    e_sq = jnp.sum(E * E, axis=0, keepdims=True)
    dist = z_sq + e_sq - 2.0 * sim
    idx = jnp.argmin(dist, axis=1)
    k_iota = jax.lax.broadcasted_iota(jnp.int32, dist.shape, 1)
    onehot = (k_iota == idx[:, None]).astype(jnp.float32)
    quant = jax.lax.dot_general(
        onehot, E, (((1,), (1,)), ((), ())),
        preferred_element_type=jnp.float32)
    q = z + (quant - z)
    hd = jnp.maximum(
        jnp.dot(q, Wd1, preferred_element_type=jnp.float32) + bd1, 0.0)
    return jnp.dot(hd, Wd2, preferred_element_type=jnp.float32) + bd2


def _body(x_ref, W1_ref, b1_ref, W2_ref, b2_ref, E_ref,
          Wd1_ref, bd1_ref, Wd2_ref, bd2_ref, out_ref):
    TB = x_ref.shape[0]
    S = TB // NSUB
    E = E_ref[...]
    W1 = W1_ref[...]
    W2 = W2_ref[...]
    Wd1 = Wd1_ref[...]
    Wd2 = Wd2_ref[...]
    b1 = b1_ref[...]
    b2 = b2_ref[...]
    bd1 = bd1_ref[...]
    bd2 = bd2_ref[...]
    for s in range(NSUB):
        rows = pl.ds(s * S, S)
        out_ref[rows, :] = _chain(
            x_ref[rows, :], W1, b1, W2, b2, E, Wd1, bd1, Wd2, bd2)


@jax.jit
def kernel(x, W1, b1, W2, b2, E, Wd1, bd1, Wd2, bd2):
    B, D = x.shape
    L, K = E.shape
    Dh = W1.shape[1]
    TB = min(2048, B)
    grid = (B // TB,)

    def batch_map(i):
        return (i, 0)

    def const_map(i):
        return (0, 0)

    full = lambda shape: pl.BlockSpec(shape, const_map)
    out = pl.pallas_call(
        _body,
        grid=grid,
        in_specs=[
            pl.BlockSpec((TB, D), batch_map),
            full((D, Dh)),
            full((1, Dh)),
            full((Dh, L)),
            full((1, L)),
            full((L, K)),
            full((L, Dh)),
            full((1, Dh)),
            full((Dh, D)),
            full((1, D)),
        ],
        out_specs=pl.BlockSpec((TB, D), batch_map),
        out_shape=jax.ShapeDtypeStruct((B, D), jnp.float32),
        compiler_params=pltpu.CompilerParams(
            dimension_semantics=("parallel",),
        ),
    )(x, W1, b1.reshape(1, -1), W2, b2.reshape(1, -1), E,
      Wd1, bd1.reshape(1, -1), Wd2, bd2.reshape(1, -1))
    return out


# speedup vs baseline: 1.0693x; 1.0185x over previous
import jax
import jax.numpy as jnp
from jax.experimental import pallas as pl
from jax.experimental.pallas import tpu as pltpu

NSUB = 2


def _body(x_ref, W1_ref, b1_ref, W2_ref, b2_ref, E_ref,
          Wd1_ref, bd1_ref, Wd2_ref, bd2_ref, out_ref):
    TB = x_ref.shape[0]
    E = E_ref[...]
    h = jnp.maximum(
        jnp.dot(x_ref[...], W1_ref[...], preferred_element_type=jnp.float32)
        + b1_ref[...], 0.0)
    z = jnp.maximum(
        jnp.dot(h, W2_ref[...], preferred_element_type=jnp.float32)
        + b2_ref[...], 0.0)
    sim = jnp.dot(z, E, preferred_element_type=jnp.float32)
    z_sq = jnp.sum(z * z, axis=1, keepdims=True)
    e_sq = jnp.sum(E * E, axis=0, keepdims=True)
    S = TB // NSUB
    for s in range(NSUB):
        r = slice(s * S, (s + 1) * S)
        dist = z_sq[r] + e_sq - 2.0 * sim[r]
        idx = jnp.argmin(dist, axis=1)
        k_iota = jax.lax.broadcasted_iota(jnp.int32, dist.shape, 1)
        onehot = (k_iota == idx[:, None]).astype(jnp.float32)
        quant = jax.lax.dot_general(
            onehot, E, (((1,), (1,)), ((), ())),
            preferred_element_type=jnp.float32)
        zc = z[r]
        q = zc + (quant - zc)
        hd = jnp.maximum(
            jnp.dot(q, Wd1_ref[...], preferred_element_type=jnp.float32)
            + bd1_ref[...], 0.0)
        out_ref[pl.ds(s * S, S), :] = (
            jnp.dot(hd, Wd2_ref[...], preferred_element_type=jnp.float32)
            + bd2_ref[...])


@jax.jit
def kernel(x, W1, b1, W2, b2, E, Wd1, bd1, Wd2, bd2):
    B, D = x.shape
    L, K = E.shape
    Dh = W1.shape[1]
    TB = min(2048, B)
    grid = (B // TB,)

    def batch_map(i):
        return (i, 0)

    def const_map(i):
        return (0, 0)

    full = lambda shape: pl.BlockSpec(shape, const_map)
    out = pl.pallas_call(
        _body,
        grid=grid,
        in_specs=[
            pl.BlockSpec((TB, D), batch_map),
            full((D, Dh)),
            full((1, Dh)),
            full((Dh, L)),
            full((1, L)),
            full((L, K)),
            full((L, Dh)),
            full((1, Dh)),
            full((Dh, D)),
            full((1, D)),
        ],
        out_specs=pl.BlockSpec((TB, D), batch_map),
        out_shape=jax.ShapeDtypeStruct((B, D), jnp.float32),
        compiler_params=pltpu.CompilerParams(
            dimension_semantics=("parallel",),
        ),
    )(x, W1, b1.reshape(1, -1), W2, b2.reshape(1, -1), E,
      Wd1, bd1.reshape(1, -1), Wd2, bd2.reshape(1, -1))
    return out
